# baseline (speedup 1.0000x reference)
import functools

import jax
import jax.numpy as jnp
from jax import lax
from jax.experimental import pallas as pl
from jax.experimental.pallas import tpu as pltpu
from jax.experimental.pallas import tpu_sc as plsc

BATCH = 4
SEQ = 2048
EMBED = 128
NW = 32
ROWS_PER_W = (BATCH * SEQ) // NW
LANES = 16


SPW = SEQ // NW


def _body(x_hbm, tok_hbm, pos_hbm, out_hbm, idx_v, tok_v, pos_v,
          g0, g1, g2, g3, p_sem, st_sem):
    cid = lax.axis_index("c")
    sid = lax.axis_index("s")
    wid = sid * 2 + cid
    s0 = wid * SPW

    g_sems = (g0, g1, g2, g3)
    idx_cps = [pltpu.async_copy(x_hbm.at[b, pl.ds(s0, SPW)], idx_v.at[b],
                                g_sems[b]) for b in range(BATCH)]
    pos_cp = pltpu.async_copy(pos_hbm.at[pl.ds(s0, SPW)], pos_v, p_sem)

    gathers = []
    for b in range(BATCH):
        idx_cps[b].wait()
        gathers.append(pltpu.async_copy(
            tok_hbm.at[idx_v.at[b]], tok_v.at[pl.ds(b * SPW, SPW)],
            g_sems[b]))

    pos_cp.wait()
    stores = []
    for b in range(BATCH):
        off = b * SPW
        gathers[b].wait()

        @plsc.parallel_loop(0, SPW, unroll=4)
        def row(r):
            for c in range(EMBED // LANES):
                sl = pl.ds(c * LANES, LANES)
                plsc.addupdate(tok_v.at[off + r, sl], pos_v[r, sl])

        stores.append(pltpu.async_copy(
            tok_v.at[pl.ds(off, SPW)],
            out_hbm.at[pl.ds(b * SEQ + s0, SPW)], st_sem))
    for st in stores:
        st.wait()


@jax.jit
def _emb_lookup(x2d, token_table, position_table):
    mesh = plsc.VectorSubcoreMesh(core_axis_name="c", subcore_axis_name="s")
    return pl.kernel(
        _body,
        mesh=mesh,
        out_type=jax.ShapeDtypeStruct((BATCH * SEQ, EMBED), jnp.float32),
        scratch_types=[
            pltpu.VMEM((BATCH, SPW), jnp.int32),
            pltpu.VMEM((ROWS_PER_W, EMBED), jnp.float32),
            pltpu.VMEM((SPW, EMBED), jnp.float32),
            pltpu.SemaphoreType.DMA,
            pltpu.SemaphoreType.DMA,
            pltpu.SemaphoreType.DMA,
            pltpu.SemaphoreType.DMA,
            pltpu.SemaphoreType.DMA,
            pltpu.SemaphoreType.DMA,
        ],
    )(x2d, token_table, position_table)


def kernel(x, token_table, position_table):
    out = _emb_lookup(x.astype(jnp.int32), token_table, position_table)
    return out.reshape(BATCH, SEQ, EMBED)


# speedup vs baseline: 1.0103x; 1.0103x over previous
import functools

import jax
import jax.numpy as jnp
from jax import lax
from jax.experimental import pallas as pl
from jax.experimental.pallas import tpu as pltpu
from jax.experimental.pallas import tpu_sc as plsc

BATCH = 4
SEQ = 2048
EMBED = 128
NW = 32
ROWS_PER_W = (BATCH * SEQ) // NW
LANES = 16


SPW = SEQ // NW


def _body(x_hbm, tok_hbm, pos_hbm, out_hbm, idx_v, tok_v, pos_v,
          g0, g1, g2, g3, p_sem, st_sem):
    cid = lax.axis_index("c")
    sid = lax.axis_index("s")
    wid = sid * 2 + cid
    s0 = wid * SPW

    g_sems = (g0, g1, g2, g3)
    idx_cps = [pltpu.async_copy(x_hbm.at[b, pl.ds(s0, SPW)], idx_v.at[b],
                                g_sems[b]) for b in range(BATCH)]
    fills = [pltpu.async_copy(pos_hbm.at[pl.ds(s0, SPW)],
                              tok_v.at[pl.ds(b * SPW, SPW)], p_sem)
             for b in range(BATCH)]
    for cp in fills:
        cp.wait()

    gathers = []
    for b in range(BATCH):
        idx_cps[b].wait()
        gathers.append(pltpu.async_copy(
            tok_hbm.at[idx_v.at[b]], tok_v.at[pl.ds(b * SPW, SPW)],
            g_sems[b], add=True))

    stores = []
    for b in range(BATCH):
        off = b * SPW
        gathers[b].wait()
        stores.append(pltpu.async_copy(
            tok_v.at[pl.ds(off, SPW)],
            out_hbm.at[pl.ds(b * SEQ + s0, SPW)], st_sem))
    for st in stores:
        st.wait()


@jax.jit
def _emb_lookup(x2d, token_table, position_table):
    mesh = plsc.VectorSubcoreMesh(core_axis_name="c", subcore_axis_name="s")
    return pl.kernel(
        _body,
        mesh=mesh,
        out_type=jax.ShapeDtypeStruct((BATCH * SEQ, EMBED), jnp.float32),
        scratch_types=[
            pltpu.VMEM((BATCH, SPW), jnp.int32),
            pltpu.VMEM((ROWS_PER_W, EMBED), jnp.float32),
            pltpu.VMEM((SPW, EMBED), jnp.float32),
            pltpu.SemaphoreType.DMA,
            pltpu.SemaphoreType.DMA,
            pltpu.SemaphoreType.DMA,
            pltpu.SemaphoreType.DMA,
            pltpu.SemaphoreType.DMA,
            pltpu.SemaphoreType.DMA,
        ],
    )(x2d, token_table, position_table)


def kernel(x, token_table, position_table):
    out = _emb_lookup(x.astype(jnp.int32), token_table, position_table)
    return out.reshape(BATCH, SEQ, EMBED)


# speedup vs baseline: 1.0114x; 1.0011x over previous
import functools

import jax
import jax.numpy as jnp
from jax import lax
from jax.experimental import pallas as pl
from jax.experimental.pallas import tpu as pltpu
from jax.experimental.pallas import tpu_sc as plsc

BATCH = 4
SEQ = 2048
EMBED = 128
NW = 32
ROWS_PER_W = (BATCH * SEQ) // NW
LANES = 16


SPW = SEQ // NW


def _body(x_hbm, tok_hbm, pos_hbm, out_hbm, idx_v, tok_v,
          g0, g1, g2, g3, p_sem, st_sem):
    cid = lax.axis_index("c")
    sid = lax.axis_index("s")
    wid = sid * 2 + cid
    s0 = wid * SPW

    g_sems = (g0, g1, g2, g3)
    idx_cps = [pltpu.async_copy(x_hbm.at[b, pl.ds(s0, SPW)], idx_v.at[b],
                                g_sems[b]) for b in range(BATCH)]
    fills = [pltpu.async_copy(pos_hbm.at[pl.ds(s0, SPW)],
                              tok_v.at[pl.ds(b * SPW, SPW)], p_sem)
             for b in range(BATCH)]
    for cp in fills:
        cp.wait()

    gathers = []
    for b in range(BATCH):
        idx_cps[b].wait()
        gathers.append(pltpu.async_copy(
            tok_hbm.at[idx_v.at[b]], tok_v.at[pl.ds(b * SPW, SPW)],
            g_sems[b], add=True))

    stores = []
    for b in range(BATCH):
        off = b * SPW
        gathers[b].wait()
        stores.append(pltpu.async_copy(
            tok_v.at[pl.ds(off, SPW)],
            out_hbm.at[pl.ds(b * SEQ + s0, SPW)], st_sem))
    for st in stores:
        st.wait()


@jax.jit
def _emb_lookup(x2d, token_table, position_table):
    mesh = plsc.VectorSubcoreMesh(core_axis_name="c", subcore_axis_name="s")
    return pl.kernel(
        _body,
        mesh=mesh,
        out_type=jax.ShapeDtypeStruct((BATCH * SEQ, EMBED), jnp.float32),
        scratch_types=[
            pltpu.VMEM((BATCH, SPW), jnp.int32),
            pltpu.VMEM((ROWS_PER_W, EMBED), jnp.float32),
            pltpu.SemaphoreType.DMA,
            pltpu.SemaphoreType.DMA,
            pltpu.SemaphoreType.DMA,
            pltpu.SemaphoreType.DMA,
            pltpu.SemaphoreType.DMA,
            pltpu.SemaphoreType.DMA,
        ],
    )(x2d, token_table, position_table)


def kernel(x, token_table, position_table):
    out = _emb_lookup(x.astype(jnp.int32), token_table, position_table)
    return out.reshape(BATCH, SEQ, EMBED)


# speedup vs baseline: 1.0604x; 1.0485x over previous
import functools

import jax
import jax.numpy as jnp
from jax import lax
from jax.experimental import pallas as pl
from jax.experimental.pallas import tpu as pltpu
from jax.experimental.pallas import tpu_sc as plsc

BATCH = 4
SEQ = 2048
EMBED = 128
NW = 32
ROWS_PER_W = (BATCH * SEQ) // NW
LANES = 16


SPW = SEQ // NW


def _body(x_hbm, tok_hbm, pos_hbm, out_hbm, idx_v, tok_v,
          g0, g1, g2, g3, p_sem, st_sem):
    cid = lax.axis_index("c")
    sid = lax.axis_index("s")
    wid = sid * 2 + cid
    s0 = wid * SPW

    g_sems = (g0, g1, g2, g3)
    idx_cps = [pltpu.async_copy(x_hbm.at[b, pl.ds(s0, SPW)], idx_v.at[b],
                                g_sems[b]) for b in range(BATCH)]
    pltpu.async_copy(pos_hbm.at[pl.ds(s0, SPW)], tok_v.at[pl.ds(0, SPW)],
                     p_sem).wait()

    gathers = [None] * BATCH
    for b in range(1, BATCH):
        off = b * SPW

        @plsc.parallel_loop(0, SPW, unroll=2)
        def cprow(r):
            for c in range(EMBED // LANES):
                sl = pl.ds(c * LANES, LANES)
                tok_v[off + r, sl] = tok_v[r, sl]

        idx_cps[b].wait()
        gathers[b] = pltpu.async_copy(
            tok_hbm.at[idx_v.at[b]], tok_v.at[pl.ds(off, SPW)],
            g_sems[b], add=True)
    idx_cps[0].wait()
    gathers[0] = pltpu.async_copy(
        tok_hbm.at[idx_v.at[0]], tok_v.at[pl.ds(0, SPW)], g_sems[0],
        add=True)

    stores = []
    for b in range(BATCH):
        off = b * SPW
        gathers[b].wait()
        stores.append(pltpu.async_copy(
            tok_v.at[pl.ds(off, SPW)],
            out_hbm.at[pl.ds(b * SEQ + s0, SPW)], st_sem))
    for st in stores:
        st.wait()


@jax.jit
def _emb_lookup(x2d, token_table, position_table):
    mesh = plsc.VectorSubcoreMesh(core_axis_name="c", subcore_axis_name="s")
    return pl.kernel(
        _body,
        mesh=mesh,
        out_type=jax.ShapeDtypeStruct((BATCH * SEQ, EMBED), jnp.float32),
        scratch_types=[
            pltpu.VMEM((BATCH, SPW), jnp.int32),
            pltpu.VMEM((ROWS_PER_W, EMBED), jnp.float32),
            pltpu.SemaphoreType.DMA,
            pltpu.SemaphoreType.DMA,
            pltpu.SemaphoreType.DMA,
            pltpu.SemaphoreType.DMA,
            pltpu.SemaphoreType.DMA,
            pltpu.SemaphoreType.DMA,
        ],
    )(x2d, token_table, position_table)


def kernel(x, token_table, position_table):
    out = _emb_lookup(x.astype(jnp.int32), token_table, position_table)
    return out.reshape(BATCH, SEQ, EMBED)
